# Initial kernel scaffold; baseline (speedup 1.0000x reference)
import jax
import jax.numpy as jnp
from jax.experimental import pallas as pl


def kernel(inp, embedding_matrix, position_embedding):
    raise NotImplementedError("write your pallas kernel here")


# speedup vs baseline: 2.4511x; 2.4511x over previous
import functools

import jax
import jax.numpy as jnp
from jax import lax
from jax.experimental import pallas as pl
from jax.experimental.pallas import tpu as pltpu
from jax.experimental.pallas import tpu_sc as plsc

BATCH = 1024
SEQLEN = 200
EMBED = 64
LANES = 16
NC = 2
NS = 16
NW = NC * NS
ROWS_PER_W = BATCH // NW
HALF = SEQLEN // 2


def _sc_embed(inp2d, table, pos):
    mesh = plsc.VectorSubcoreMesh(core_axis_name="c", subcore_axis_name="s")

    @functools.partial(
        pl.kernel,
        out_type=jax.ShapeDtypeStruct((BATCH, SEQLEN, EMBED), jnp.float32),
        mesh=mesh,
        scratch_types=[
            pltpu.VMEM((2, HALF), jnp.int32),
            pltpu.VMEM((SEQLEN, EMBED), jnp.float32),
            pltpu.VMEM((SEQLEN, EMBED), jnp.float32),
            pltpu.SemaphoreType.DMA,
        ],
        compiler_params=pltpu.CompilerParams(use_tc_tiling_on_sc=False),
    )
    def k(inp_hbm, table_hbm, pos_hbm, out_hbm, idx_v, rows_v, pos_v, sem):
        wid = lax.axis_index("s") * NC + lax.axis_index("c")
        base = wid * ROWS_PER_W
        pltpu.sync_copy(pos_hbm, pos_v)

        def body(i, carry):
            b = base + i
            pltpu.sync_copy(inp_hbm.at[b], idx_v)
            pltpu.async_copy(
                table_hbm.at[idx_v.at[0]], rows_v.at[pl.ds(0, HALF)], sem
            ).wait()
            pltpu.async_copy(
                table_hbm.at[idx_v.at[1]], rows_v.at[pl.ds(HALF, HALF)], sem
            ).wait()

            def add_row(r, c):
                for c4 in range(EMBED // LANES):
                    sl = pl.ds(c4 * LANES, LANES)
                    rows_v[r, sl] = rows_v[r, sl] + pos_v[r, sl]
                return c

            lax.fori_loop(0, SEQLEN, add_row, 0)
            pltpu.sync_copy(rows_v, out_hbm.at[b])
            return carry

        lax.fori_loop(0, ROWS_PER_W, body, 0)

    return k(inp2d, table, pos)


def kernel(inp, embedding_matrix, position_embedding):
    inp2d = inp.astype(jnp.int32).reshape(BATCH, 2, HALF)
    return _sc_embed(inp2d, embedding_matrix, position_embedding)


# speedup vs baseline: 3.0754x; 1.2547x over previous
import functools

import jax
import jax.numpy as jnp
from jax import lax
from jax.experimental import pallas as pl
from jax.experimental.pallas import tpu as pltpu
from jax.experimental.pallas import tpu_sc as plsc

BATCH = 1024
SEQLEN = 200
EMBED = 64
LANES = 16
NC = 2
NS = 16
NW = NC * NS
ROWS_PER_W = BATCH // NW
HALF = SEQLEN // 2


def _sc_embed(inp3d, table, pos):
    mesh = plsc.VectorSubcoreMesh(core_axis_name="c", subcore_axis_name="s")

    @functools.partial(
        pl.kernel,
        out_type=jax.ShapeDtypeStruct((BATCH, SEQLEN, EMBED), jnp.float32),
        mesh=mesh,
        scratch_types=[
            pltpu.VMEM((2 * ROWS_PER_W, HALF), jnp.int32),
            pltpu.VMEM((SEQLEN, EMBED), jnp.float32),
            pltpu.VMEM((SEQLEN, EMBED), jnp.float32),
            pltpu.VMEM((SEQLEN, EMBED), jnp.float32),
            pltpu.SemaphoreType.DMA,
            pltpu.SemaphoreType.DMA,
            pltpu.SemaphoreType.DMA,
            pltpu.SemaphoreType.DMA,
        ],
        compiler_params=pltpu.CompilerParams(use_tc_tiling_on_sc=False),
    )
    def k(inp_hbm, table_hbm, pos_hbm, out_hbm,
          idx_v, rows0, rows1, pos_v, g0, g1, s0, s1):
        wid = lax.axis_index("s") * NC + lax.axis_index("c")
        base = wid * ROWS_PER_W
        bufs = (rows0, rows1)
        gsems = (g0, g1)
        ssems = (s0, s1)

        pltpu.sync_copy(inp_hbm.at[wid], idx_v)
        pltpu.sync_copy(pos_hbm, pos_v)

        def fire_gather(r, p):
            d0 = pltpu.async_copy(
                table_hbm.at[idx_v.at[2 * r]],
                bufs[p].at[pl.ds(0, HALF)], gsems[p])
            d1 = pltpu.async_copy(
                table_hbm.at[idx_v.at[2 * r + 1]],
                bufs[p].at[pl.ds(HALF, HALF)], gsems[p])
            return d0, d1

        def add_pos(p):
            buf = bufs[p]

            @plsc.parallel_loop(0, SEQLEN, 1, unroll=4)
            def _(r):
                for c4 in range(EMBED // LANES):
                    sl = pl.ds(c4 * LANES, LANES)
                    buf[r, sl] = buf[r, sl] + pos_v[r, sl]

        gd = [None, None]
        sd = [None, None]
        gd[0] = fire_gather(0, 0)
        for r in range(ROWS_PER_W):
            p = r % 2
            q = 1 - p
            if r + 1 < ROWS_PER_W:
                if sd[q] is not None:
                    sd[q].wait()
                gd[q] = fire_gather(r + 1, q)
            gd[p][0].wait()
            gd[p][1].wait()
            add_pos(p)
            sd[p] = pltpu.async_copy(bufs[p], out_hbm.at[base + r], ssems[p])
        sd[0].wait()
        sd[1].wait()

    return k(inp3d, table, pos)


def kernel(inp, embedding_matrix, position_embedding):
    inp3d = inp.astype(jnp.int32).reshape(NW, 2 * ROWS_PER_W, HALF)
    return _sc_embed(inp3d, embedding_matrix, position_embedding)
